# Initial kernel scaffold; baseline (speedup 1.0000x reference)
import jax
import jax.numpy as jnp
from jax.experimental import pallas as pl


def kernel(soft_perm):
    raise NotImplementedError("write your pallas kernel here")


# speedup vs baseline: 81.1363x; 81.1363x over previous
import functools

import jax
import jax.numpy as jnp
from jax import lax
from jax.experimental import pallas as pl
from jax.experimental.pallas import tpu as pltpu
from jax.experimental.pallas import tpu_sc as plsc

_B, _N = 64, 256
_L = 16
_NCH = _N // _L
_NEG = float(jnp.finfo(jnp.float32).min)
_NUM_CORES = 2
_NUM_SUBCORES = 16
_PER_WORKER = _B // (_NUM_CORES * _NUM_SUBCORES)


def _greedy_body(inp_hbm, out_hbm, a_ref, row_max, row_arg, colneg, sem):
    lanes = lax.iota(jnp.int32, _L)
    lane0 = lanes == 0
    zeros_f = jnp.zeros((_L,), jnp.float32)
    neg_f = jnp.full((_L,), _NEG, jnp.float32)
    ones_f = jnp.ones((_L,), jnp.float32)

    wid = lax.axis_index("s") * _NUM_CORES + lax.axis_index("c")

    def rowpass(r):
        base = jnp.full((_L,), r * _N, jnp.int32)
        bv = neg_f
        bc = jnp.zeros((_L,), jnp.int32)
        for j in range(_NCH):
            col = j * _L + lanes
            av = plsc.load_gather(a_ref, [base + col])
            v = av + colneg[pl.ds(j * _L, _L)]
            upd = v > bv
            bv = jnp.where(upd, v, bv)
            bc = jnp.where(upd, col, bc)
        m = jnp.max(bv)
        c = jnp.min(jnp.where(bv >= m, bc, _N))
        return m, c

    for k in range(_PER_WORKER):
        b = wid * _PER_WORKER + k
        pltpu.async_copy(inp_hbm.at[b], a_ref, sem).wait()

        for j in range(_NCH):
            colneg[pl.ds(j * _L, _L)] = zeros_f

        def init_row(r, carry):
            m, c = rowpass(r)
            rvec = jnp.full((_L,), r, jnp.int32)
            plsc.store_scatter(row_max, [rvec], jnp.full((_L,), m), mask=lane0)
            plsc.store_scatter(row_arg, [rvec], jnp.full((_L,), c), mask=lane0)
            return carry

        lax.fori_loop(0, _N, init_row, 0)

        def step(i, carry):
            def not_done(st):
                return st == jnp.int32(0)

            def attempt(st):
                bv = neg_f
                br = jnp.zeros((_L,), jnp.int32)
                for j in range(_NCH):
                    rows = j * _L + lanes
                    v = row_max[pl.ds(j * _L, _L)]
                    upd = v > bv
                    bv = jnp.where(upd, v, bv)
                    br = jnp.where(upd, rows, br)
                m = jnp.max(bv)
                r = jnp.min(jnp.where(bv >= m, br, _N))
                rvec = jnp.full((_L,), r, jnp.int32)
                cvec = plsc.load_gather(row_arg, [rvec])
                cmask_v = plsc.load_gather(colneg, [cvec])
                ok = jnp.min(cmask_v) == jnp.float32(0.0)

                @pl.when(ok)
                def _assign():
                    plsc.store_scatter(colneg, [cvec], neg_f, mask=lane0)
                    plsc.store_scatter(row_max, [rvec], neg_f, mask=lane0)

                @pl.when(jnp.logical_not(ok))
                def _refresh():
                    nm, nc = rowpass(r)
                    plsc.store_scatter(
                        row_max, [rvec], jnp.full((_L,), nm), mask=lane0)
                    plsc.store_scatter(
                        row_arg, [rvec], jnp.full((_L,), nc), mask=lane0)

                return jnp.where(ok, jnp.int32(1), jnp.int32(0))

            lax.while_loop(not_done, attempt, jnp.int32(0))
            return carry

        lax.fori_loop(0, _N, step, 0)

        def zero_row(r, carry):
            base = jnp.full((_L,), r * _N, jnp.int32)
            for j in range(_NCH):
                plsc.store_scatter(a_ref, [base + j * _L + lanes], zeros_f)
            return carry

        lax.fori_loop(0, _N, zero_row, 0)
        for j in range(_NCH):
            rows = j * _L + lanes
            cols = row_arg[pl.ds(j * _L, _L)]
            plsc.store_scatter(a_ref, [rows * _N + cols], ones_f)

        pltpu.async_copy(a_ref, out_hbm.at[b], sem).wait()


@jax.jit
def _greedy_hard_perm_sc(soft_perm):
    mesh = plsc.VectorSubcoreMesh(
        core_axis_name="c", subcore_axis_name="s",
        num_cores=_NUM_CORES, num_subcores=_NUM_SUBCORES)
    out = pl.kernel(
        _greedy_body,
        out_type=jax.ShapeDtypeStruct((_B, _N * _N), jnp.float32),
        mesh=mesh,
        compiler_params=pltpu.CompilerParams(needs_layout_passes=False),
        scratch_types=[
            pltpu.VMEM((_N * _N,), jnp.float32),
            pltpu.VMEM((_N,), jnp.float32),
            pltpu.VMEM((_N,), jnp.int32),
            pltpu.VMEM((_N,), jnp.float32),
            pltpu.SemaphoreType.DMA,
        ],
    )(soft_perm.reshape(_B, _N * _N))
    return out.reshape(_B, _N, _N)


def kernel(soft_perm):
    hard = lax.stop_gradient(_greedy_hard_perm_sc(soft_perm))
    return hard + (soft_perm - lax.stop_gradient(soft_perm))
